# Initial kernel scaffold; baseline (speedup 1.0000x reference)
import jax
import jax.numpy as jnp
from jax.experimental import pallas as pl


def kernel(x_qkv, batch_size, seq_len, input_ids, position_ids, word_table, pos_table):
    raise NotImplementedError("write your pallas kernel here")


# speedup vs baseline: 1.6694x; 1.6694x over previous
import functools

import jax
import jax.numpy as jnp
from jax import lax
from jax.experimental import pallas as pl
from jax.experimental.pallas import tpu as pltpu
from jax.experimental.pallas import tpu_sc as plsc

_NC = 2
_NS = 16
_NW = _NC * _NS
_CHUNK = 128


def _embed_lookup(ids2d, pids2d, word_table, pos_table):
    n_chunks, chunk = ids2d.shape
    d = word_table.shape[1]
    cpw = n_chunks // _NW
    n_total = n_chunks * chunk
    mesh = plsc.VectorSubcoreMesh(core_axis_name="c", subcore_axis_name="s")

    @functools.partial(
        pl.kernel,
        out_type=jax.ShapeDtypeStruct((n_total, d), jnp.float32),
        mesh=mesh,
        scratch_types=[
            pltpu.VMEM((cpw, chunk), jnp.int32),
            pltpu.VMEM((cpw, chunk), jnp.int32),
            pltpu.VMEM((cpw, chunk, d), jnp.float32),
            pltpu.VMEM((cpw, chunk, d), jnp.float32),
            pltpu.SemaphoreType.DMA,
            pltpu.SemaphoreType.DMA,
        ],
    )
    def k(ids_hbm, pids_hbm, wt_hbm, pt_hbm, out_hbm,
          widx, pidx, wrows, prows, wsem, psem):
        wid = lax.axis_index("s") * _NC + lax.axis_index("c")
        c0 = wid * cpw
        pltpu.sync_copy(ids_hbm.at[pl.ds(c0, cpw)], widx)
        pltpu.sync_copy(pids_hbm.at[pl.ds(c0, cpw)], pidx)
        copies = []
        for j in range(cpw):
            copies.append(pltpu.async_copy(wt_hbm.at[widx.at[j]], wrows.at[j], wsem))
            copies.append(pltpu.async_copy(pt_hbm.at[pidx.at[j]], prows.at[j], psem))
        for c in copies:
            c.wait()

        def row_body(r, carry):
            for j in range(cpw):
                for v in range(d // 16):
                    sl = pl.ds(v * 16, 16)
                    wrows[j, r, sl] += prows[j, r, sl]
            return carry

        lax.fori_loop(0, chunk, row_body, 0)
        base = c0 * chunk
        for j in range(cpw):
            pltpu.sync_copy(wrows.at[j], out_hbm.at[pl.ds(base + j * chunk, chunk)])

    return k(ids2d, pids2d, word_table, pos_table)


def kernel(x_qkv, batch_size, seq_len, input_ids, position_ids, word_table, pos_table):
    b, s = input_ids.shape
    d = word_table.shape[1]
    n = b * s
    ids2d = input_ids.reshape(n // _CHUNK, _CHUNK)
    pids2d = position_ids.reshape(n // _CHUNK, _CHUNK)
    out = _embed_lookup(ids2d, pids2d, word_table, pos_table)
    return out.reshape(b, s, d)
